# Initial kernel scaffold; baseline (speedup 1.0000x reference)
import functools

import jax
import jax.numpy as jnp
from jax import lax
from jax.experimental import pallas as pl
from jax.experimental.pallas import tpu as pltpu
from jax.experimental.pallas import tpu_sc as plsc

_D = 16
_B = 4096
_V = 16384
_F = 26
_FP = 32
_NW = 32
_CH = 4
_UB = _B // _NW
_IB = _V // _NW


@functools.partial(
    pl.kernel,
    mesh=plsc.VectorSubcoreMesh(core_axis_name="c", subcore_axis_name="s"),
    out_type=[
        jax.ShapeDtypeStruct((_B, 32), jnp.float32),
        jax.ShapeDtypeStruct((_V, 32), jnp.float32),
    ],
    scratch_types=[
        pltpu.VMEM((_IB * _F,), jnp.int32),
        pltpu.VMEM((_IB * _FP,), jnp.int32),
        pltpu.VMEM((_CH * _F, _D), jnp.float32),
        pltpu.VMEM((_CH * _FP,), jnp.float32),
        pltpu.VMEM((_IB, 32), jnp.float32),
        pltpu.VMEM((32,), jnp.float32),
        pltpu.SemaphoreType.DMA,
        pltpu.SemaphoreType.DMA,
    ],
)
def _fm_sc(ue_hbm, ie_hbm, uidx_hbm, uwidx_hbm, iidx_hbm, iwidx_hbm,
           wu_hbm, wi_hbm, par_hbm,
           p_hbm, q_hbm,
           idx_v, widx_v, rows_v, wrows_v, out_v, par_v, sem0, sem1):
    wid = lax.axis_index("s") * 2 + lax.axis_index("c")
    pltpu.sync_copy(par_hbm, par_v)
    h1v = par_v[pl.ds(0, 16)]
    bias16 = par_v[pl.ds(16, 16)]
    lane = lax.iota(jnp.int32, 16)
    m10 = lane < 10
    m0 = lane == 0
    m1 = lane == 1
    zero = jnp.zeros((16,), jnp.float32)

    def process(tab_hbm, idx_hbm, widx_hbm, w_hbm, out_hbm, n_tile, is_user):
        ni = n_tile * _F
        nw = n_tile * _FP
        pltpu.sync_copy(idx_hbm.at[pl.ds(wid * ni, ni)], idx_v.at[pl.ds(0, ni)])
        pltpu.sync_copy(widx_hbm.at[pl.ds(wid * nw, nw)], widx_v.at[pl.ds(0, nw)])

        def body(c, carry):
            cp0 = pltpu.async_copy(
                tab_hbm.at[idx_v.at[pl.ds(c * (_CH * _F), _CH * _F)]], rows_v, sem0)
            cp1 = pltpu.async_copy(
                w_hbm.at[widx_v.at[pl.ds(c * (_CH * _FP), _CH * _FP)]], wrows_v, sem1)
            cp0.wait()
            cp1.wait()
            for k in range(_CH):
                b = k * _F
                r = rows_v[b, :]
                s = r
                q = r * r
                for j in range(1, _F):
                    r = rows_v[b + j, :]
                    s = s + r
                    q = q + r * r
                tv = (0.5 * (s * s - q)) * h1v
                wv1 = wrows_v[pl.ds(k * _FP, 16)]
                wv2 = jnp.where(m10, wrows_v[pl.ds(k * _FP + 16, 16)], zero)
                tv = tv + wv1 + wv2
                if is_user:
                    tv = tv + bias16
                sc = jnp.sum(tv)
                if is_user:
                    tail = jnp.where(m0, sc, jnp.where(m1, 1.0, 0.0))
                else:
                    tail = jnp.where(m0, 1.0, jnp.where(m1, sc, 0.0))
                e = c * _CH + k
                out_v[e, pl.ds(0, 16)] = s
                out_v[e, pl.ds(16, 16)] = tail
            return carry

        lax.fori_loop(0, n_tile // _CH, body, 0)
        pltpu.sync_copy(out_v.at[pl.ds(0, n_tile)],
                        out_hbm.at[pl.ds(wid * n_tile, n_tile)])

    process(ue_hbm, uidx_hbm, uwidx_hbm, wu_hbm, p_hbm, _UB, True)
    process(ie_hbm, iidx_hbm, iwidx_hbm, wi_hbm, q_hbm, _IB, False)


def kernel(user_features_embeddings, item_features_embeddings,
           user_feature_indexes, item_feature_indexes,
           w_user, w_item, global_bias, h1, h2):
    ui = user_feature_indexes.astype(jnp.int32)
    ii = item_feature_indexes.astype(jnp.int32)
    uif = ui.reshape(-1)
    iif = ii.reshape(-1)
    uwp = jnp.pad(ui, ((0, 0), (0, _FP - _F))).reshape(-1)
    iwp = jnp.pad(ii, ((0, 0), (0, _FP - _F))).reshape(-1)
    par = jnp.concatenate([
        h1.astype(jnp.float32),
        jnp.full((16,), global_bias[0] / 16.0, dtype=jnp.float32),
    ])
    p32, q32 = _fm_sc(user_features_embeddings, item_features_embeddings,
                      uif, uwp, iif, iwp,
                      w_user, w_item, par)
    return p32[:, :18], q32[:, :18], h2


# speedup vs baseline: 4.9282x; 4.9282x over previous
import functools

import jax
import jax.numpy as jnp
from jax import lax
from jax.experimental import pallas as pl
from jax.experimental.pallas import tpu as pltpu
from jax.experimental.pallas import tpu_sc as plsc

_D = 16
_B = 4096
_V = 16384
_F = 26
_FP = 32
_NW = 32
_CH = 16
_SG = 4
_UB = _B // _NW
_IB = _V // _NW


@functools.partial(
    pl.kernel,
    mesh=plsc.VectorSubcoreMesh(core_axis_name="c", subcore_axis_name="s"),
    compiler_params=pltpu.CompilerParams(needs_layout_passes=False,
                                         use_tc_tiling_on_sc=False),
    out_type=[
        jax.ShapeDtypeStruct((_B, 32), jnp.float32),
        jax.ShapeDtypeStruct((_V, 32), jnp.float32),
    ],
    scratch_types=[
        pltpu.VMEM((_IB * _F,), jnp.int32),
        pltpu.VMEM((_IB * _FP,), jnp.int32),
        pltpu.VMEM((_CH * _F, _D), jnp.float32),
        pltpu.VMEM((_CH * _FP,), jnp.float32),
        pltpu.VMEM((_CH * _D,), jnp.float32),
        pltpu.VMEM((_IB, 32), jnp.float32),
        pltpu.VMEM((32,), jnp.float32),
        pltpu.SemaphoreType.DMA,
        pltpu.SemaphoreType.DMA,
    ],
)
def _fm_sc(ue_hbm, ie_hbm, uidx_hbm, uwidx_hbm, iidx_hbm, iwidx_hbm,
           wu_hbm, wi_hbm, par_hbm,
           p_hbm, q_hbm,
           idx_v, widx_v, rows_v, wrows_v, tv_v, out_v, par_v, sem0, sem1):
    wid = lax.axis_index("s") * 2 + lax.axis_index("c")
    pltpu.sync_copy(par_hbm, par_v)
    h1v = par_v[pl.ds(0, 16)]
    bias16 = par_v[pl.ds(16, 16)]
    lane = lax.iota(jnp.int32, 16)
    m10 = lane < 10
    zero = jnp.zeros((16,), jnp.float32)
    ones = jnp.ones((16,), jnp.float32)
    tpose_base = lane * _D

    def process(tab_hbm, idx_hbm, widx_hbm, w_hbm, out_hbm, n_tile, is_user):
        ni = n_tile * _F
        nw = n_tile * _FP
        pltpu.sync_copy(idx_hbm.at[pl.ds(wid * ni, ni)], idx_v.at[pl.ds(0, ni)])
        pltpu.sync_copy(widx_hbm.at[pl.ds(wid * nw, nw)], widx_v.at[pl.ds(0, nw)])

        def body(c, carry):
            cps = []
            for t in range(_SG):
                nrow = _CH * _F // _SG
                nwv = _CH * _FP // _SG
                cps.append(pltpu.async_copy(
                    tab_hbm.at[idx_v.at[pl.ds(c * (_CH * _F) + t * nrow, nrow)]],
                    rows_v.at[pl.ds(t * nrow, nrow)], sem0))
                cps.append(pltpu.async_copy(
                    w_hbm.at[widx_v.at[pl.ds(c * (_CH * _FP) + t * nwv, nwv)]],
                    wrows_v.at[pl.ds(t * nwv, nwv)], sem1))
            for cp in cps:
                cp.wait()
            for k in range(_CH):
                b = k * _F
                r = rows_v[b, :]
                s = r
                q = r * r
                for j in range(1, _F):
                    r = rows_v[b + j, :]
                    s = s + r
                    q = q + r * r
                tv = (0.5 * (s * s - q)) * h1v
                wv1 = wrows_v[pl.ds(k * _FP, 16)]
                wv2 = jnp.where(m10, wrows_v[pl.ds(k * _FP + 16, 16)], zero)
                tv = tv + wv1 + wv2
                if is_user:
                    tv = tv + bias16
                tv_v[pl.ds(k * _D, 16)] = tv
                out_v[c * _CH + k, pl.ds(0, 16)] = s
            svec = plsc.load_gather(tv_v, [tpose_base])
            for d in range(1, _D):
                svec = svec + plsc.load_gather(tv_v, [tpose_base + d])
            row_idx = c * _CH + lane
            if is_user:
                plsc.store_scatter(out_v, [row_idx, lane * 0 + 16], svec)
                plsc.store_scatter(out_v, [row_idx, lane * 0 + 17], ones)
            else:
                plsc.store_scatter(out_v, [row_idx, lane * 0 + 16], ones)
                plsc.store_scatter(out_v, [row_idx, lane * 0 + 17], svec)
            return carry

        lax.fori_loop(0, n_tile // _CH, body, 0)
        pltpu.sync_copy(out_v.at[pl.ds(0, n_tile)],
                        out_hbm.at[pl.ds(wid * n_tile, n_tile)])

    process(ue_hbm, uidx_hbm, uwidx_hbm, wu_hbm, p_hbm, _UB, True)
    process(ie_hbm, iidx_hbm, iwidx_hbm, wi_hbm, q_hbm, _IB, False)


def kernel(user_features_embeddings, item_features_embeddings,
           user_feature_indexes, item_feature_indexes,
           w_user, w_item, global_bias, h1, h2):
    ui = user_feature_indexes.astype(jnp.int32)
    ii = item_feature_indexes.astype(jnp.int32)
    uif = ui.reshape(-1)
    iif = ii.reshape(-1)
    uwp = jnp.pad(ui, ((0, 0), (0, _FP - _F))).reshape(-1)
    iwp = jnp.pad(ii, ((0, 0), (0, _FP - _F))).reshape(-1)
    par = jnp.concatenate([
        h1.astype(jnp.float32),
        jnp.full((16,), global_bias[0] / 16.0, dtype=jnp.float32),
    ])
    p32, q32 = _fm_sc(user_features_embeddings, item_features_embeddings,
                      uif, uwp, iif, iwp,
                      w_user, w_item, par)
    return p32[:, :18], q32[:, :18], h2
